# Initial kernel scaffold; baseline (speedup 1.0000x reference)
import jax
import jax.numpy as jnp
from jax.experimental import pallas as pl


def kernel(feat, edge_index, edge_affine, W, b):
    raise NotImplementedError("write your pallas kernel here")


# speedup vs baseline: 3.8089x; 3.8089x over previous
import functools

import jax
import jax.numpy as jnp
from jax import lax
from jax.experimental import pallas as pl
from jax.experimental.pallas import tpu as pltpu
from jax.experimental.pallas import tpu_sc as plsc

_NW = 32
_CHUNK = 128


def _sc_partial_agg(feat, src2d, dst2d, aff2d):
    n, d = feat.shape
    n_pad = ((n + 639) // 640) * 640
    n_chunks = src2d.shape[0]
    cpw = n_chunks // _NW
    rpt = n_pad // 16
    mesh = plsc.VectorSubcoreMesh(core_axis_name="c", subcore_axis_name="s")

    @functools.partial(
        pl.kernel,
        out_type=jax.ShapeDtypeStruct((2, n_pad, d), jnp.float32),
        mesh=mesh,
        scratch_types=[
            pltpu.VMEM((cpw, _CHUNK), jnp.int32),
            pltpu.VMEM((cpw, _CHUNK), jnp.int32),
            pltpu.VMEM((cpw, _CHUNK), jnp.float32),
            pltpu.VMEM((_CHUNK, d), jnp.float32),
            pltpu.VMEM_SHARED((n_pad, d), jnp.float32),
            pltpu.SemaphoreType.DMA,
        ],
    )
    def k(feat_hbm, src_hbm, dst_hbm, aff_hbm, out_hbm,
          src_v, dst_v, aff_v, rows_v, agg_sh, sem):
        cid = lax.axis_index("c")
        sid = lax.axis_index("s")
        wid = cid * 16 + sid

        def zrow(r, carry):
            for g in range(8):
                rows_v[r, pl.ds(g * 16, 16)] = jnp.zeros((16,), jnp.float32)
            return carry
        lax.fori_loop(0, _CHUNK, zrow, 0)
        for i in range(rpt // _CHUNK):
            pltpu.sync_copy(rows_v,
                            agg_sh.at[pl.ds(sid * rpt + i * _CHUNK, _CHUNK)])
        plsc.subcore_barrier()

        pltpu.sync_copy(src_hbm.at[pl.ds(wid * cpw, cpw)], src_v)
        pltpu.sync_copy(dst_hbm.at[pl.ds(wid * cpw, cpw)], dst_v)
        pltpu.sync_copy(aff_hbm.at[pl.ds(wid * cpw, cpw)], aff_v)

        def chunk_body(j, carry):
            pltpu.async_copy(feat_hbm.at[src_v.at[j]], rows_v, sem).wait()

            def edge_body(eg, c2):
                a16 = aff_v[j, pl.ds(eg * 16, 16)]
                for l in range(16):
                    av = jnp.full((16,), a16[l], jnp.float32)
                    e = eg * 16 + l
                    for g in range(8):
                        sl = pl.ds(g * 16, 16)
                        rows_v[e, sl] = rows_v[e, sl] * av
                return c2
            lax.fori_loop(0, _CHUNK // 16, edge_body, 0)

            pltpu.sync_copy(rows_v, agg_sh.at[dst_v.at[j]], add=True)
            return carry
        lax.fori_loop(0, cpw, chunk_body, 0)

        plsc.subcore_barrier()
        pltpu.sync_copy(agg_sh.at[pl.ds(sid * rpt, rpt)],
                        out_hbm.at[cid, pl.ds(sid * rpt, rpt)])

    return k(feat, src2d, dst2d, aff2d)


def _tc_combine(feat, aggs, wt, b2d):
    n, d = feat.shape
    d_out = wt.shape[1]
    blk = 2000

    def body(x_ref, a_ref, w_ref, b_ref, o_ref):
        a = a_ref[0] + a_ref[1]
        w1 = w_ref[pl.ds(0, d), :]
        w2 = w_ref[pl.ds(d, d), :]
        o_ref[...] = (
            jnp.dot(x_ref[...], w1, preferred_element_type=jnp.float32)
            + jnp.dot(a, w2, preferred_element_type=jnp.float32)
            + b_ref[...])

    return pl.pallas_call(
        body,
        grid=(n // blk,),
        in_specs=[
            pl.BlockSpec((blk, d), lambda i: (i, 0)),
            pl.BlockSpec((2, blk, d), lambda i: (0, i, 0)),
            pl.BlockSpec((2 * d, d_out), lambda i: (0, 0)),
            pl.BlockSpec((1, d_out), lambda i: (0, 0)),
        ],
        out_specs=pl.BlockSpec((blk, d_out), lambda i: (i, 0)),
        out_shape=jax.ShapeDtypeStruct((n, d_out), jnp.float32),
    )(feat, aggs, wt, b2d)


def kernel(feat, edge_index, edge_affine, W, b):
    n, d = feat.shape
    e = edge_index.shape[1]
    unit = _NW * _CHUNK * 8
    e_pad = ((e + unit - 1) // unit) * unit
    pad = e_pad - e

    src = jnp.concatenate([edge_index[0], jnp.zeros((pad,), jnp.int32)])
    dst = jnp.concatenate([edge_index[1], jnp.zeros((pad,), jnp.int32)])
    aff = jnp.concatenate([edge_affine, jnp.zeros((pad,), jnp.float32)])
    src2d = src.reshape(e_pad // _CHUNK, _CHUNK)
    dst2d = dst.reshape(e_pad // _CHUNK, _CHUNK)
    aff2d = aff.reshape(e_pad // _CHUNK, _CHUNK)

    aggs = _sc_partial_agg(feat, src2d, dst2d, aff2d)
    return _tc_combine(feat, aggs, W.T, b[None, :])


# speedup vs baseline: 5.2849x; 1.3875x over previous
import functools

import jax
import jax.numpy as jnp
from jax import lax
from jax.experimental import pallas as pl
from jax.experimental.pallas import tpu as pltpu
from jax.experimental.pallas import tpu_sc as plsc

_CHUNK = 128
_NBUF = 4
_NPASS = 2


def _sc_partial_agg(feat_cat, src2d, dst2d, aff2d, n, n_pad):
    dh = feat_cat.shape[1]
    n_chunks = src2d.shape[0]
    cpt = n_chunks // 16
    cpp = cpt // _NPASS
    rpt = n_pad // 16
    mesh = plsc.VectorSubcoreMesh(core_axis_name="c", subcore_axis_name="s")

    @functools.partial(
        pl.kernel,
        out_type=jax.ShapeDtypeStruct((2, n_pad, dh), jnp.float32),
        mesh=mesh,
        compiler_params=pltpu.CompilerParams(use_tc_tiling_on_sc=False),
        scratch_types=[
            pltpu.VMEM((cpp, _CHUNK), jnp.int32),
            pltpu.VMEM((cpp, _CHUNK), jnp.int32),
            pltpu.VMEM((cpp, _CHUNK), jnp.float32),
            pltpu.VMEM((_NBUF, _CHUNK, dh), jnp.float32),
            pltpu.VMEM_SHARED((n_pad, dh), jnp.float32),
            pltpu.SemaphoreType.DMA((_NBUF,)),
            pltpu.SemaphoreType.DMA((_NBUF,)),
        ],
    )
    def k(feat_hbm, src_hbm, dst_hbm, aff_hbm, out_hbm,
          src_v, dst_v, aff_v, rows_v, agg_sh, sg, ss):
        cid = lax.axis_index("c")
        sid = lax.axis_index("s")
        ngrp = dh // 16

        zbuf = rows_v.at[0]

        def zrow(r, carry):
            for g in range(ngrp):
                zbuf[r, pl.ds(g * 16, 16)] = jnp.zeros((16,), jnp.float32)
            return carry
        lax.fori_loop(0, _CHUNK, zrow, 0)
        for i in range(rpt // _CHUNK):
            pltpu.sync_copy(zbuf,
                            agg_sh.at[pl.ds(sid * rpt + i * _CHUNK, _CHUNK)])
        plsc.subcore_barrier()

        def _gather(j, b):
            pltpu.async_copy(feat_hbm.at[src_v.at[j]], rows_v.at[b], sg.at[b])

        def _scale(j, b):
            buf = rows_v.at[b]

            def edge_body(eg, c2):
                a16 = aff_v[j, pl.ds(eg * 16, 16)]
                for l in range(16):
                    av = jnp.full((16,), a16[l], jnp.float32)
                    e = eg * 16 + l
                    for g in range(ngrp):
                        sl = pl.ds(g * 16, 16)
                        buf[e, sl] = buf[e, sl] * av
                return c2
            lax.fori_loop(0, _CHUNK // 16, edge_body, 0)

        for p in range(_NPASS):
            base = sid * cpt + p * cpp
            pltpu.sync_copy(src_hbm.at[pl.ds(base, cpp)], src_v)
            pltpu.sync_copy(dst_hbm.at[pl.ds(base, cpp)], dst_v)
            pltpu.sync_copy(aff_hbm.at[pl.ds(base, cpp)], aff_v)
            off = jnp.full((16,), cid * n, jnp.int32)

            def adj(r, carry):
                for g in range(_CHUNK // 16):
                    sl = pl.ds(g * 16, 16)
                    src_v[r, sl] = src_v[r, sl] + off
                return carry
            lax.fori_loop(0, cpp, adj, 0)

            _gather(0, 0)
            _gather(1, 1)

            def chunk_body(i, carry):
                for u in range(_NBUF):
                    j = i * _NBUF + u
                    b = u
                    bn = (u + 2) % _NBUF
                    pltpu.make_async_copy(feat_hbm.at[src_v.at[j]],
                                          rows_v.at[b], sg.at[b]).wait()
                    _scale(j, b)
                    pltpu.async_copy(rows_v.at[b], agg_sh.at[dst_v.at[j]],
                                     ss.at[b], add=True)

                    @pl.when(j >= 2)
                    def _drain():
                        pltpu.make_async_copy(rows_v.at[bn],
                                              agg_sh.at[dst_v.at[j]],
                                              ss.at[bn]).wait()

                    @pl.when(j + 2 < cpp)
                    def _refill():
                        _gather(j + 2, bn)
                return carry
            lax.fori_loop(0, cpp // _NBUF, chunk_body, 0)

            for b in (2, 3):
                pltpu.make_async_copy(rows_v.at[b], agg_sh.at[dst_v.at[0]],
                                      ss.at[b]).wait()

        plsc.subcore_barrier()
        pltpu.sync_copy(agg_sh.at[pl.ds(sid * rpt, rpt)],
                        out_hbm.at[cid, pl.ds(sid * rpt, rpt)])

    return k(feat_cat, src2d, dst2d, aff2d)


def _tc_combine(feat, aggs, wt, b2d):
    n, d = feat.shape
    dh = d // 2
    d_out = wt.shape[1]
    blk = 2000

    def body(x_ref, a_ref, w_ref, b_ref, o_ref):
        w1 = w_ref[pl.ds(0, d), :]
        w2a = w_ref[pl.ds(d, dh), :]
        w2b = w_ref[pl.ds(d + dh, dh), :]
        o_ref[...] = (
            jnp.dot(x_ref[...], w1, preferred_element_type=jnp.float32)
            + jnp.dot(a_ref[0], w2a, preferred_element_type=jnp.float32)
            + jnp.dot(a_ref[1], w2b, preferred_element_type=jnp.float32)
            + b_ref[...])

    return pl.pallas_call(
        body,
        grid=(n // blk,),
        in_specs=[
            pl.BlockSpec((blk, d), lambda i: (i, 0)),
            pl.BlockSpec((2, blk, dh), lambda i: (0, i, 0)),
            pl.BlockSpec((2 * d, d_out), lambda i: (0, 0)),
            pl.BlockSpec((1, d_out), lambda i: (0, 0)),
        ],
        out_specs=pl.BlockSpec((blk, d_out), lambda i: (i, 0)),
        out_shape=jax.ShapeDtypeStruct((n, d_out), jnp.float32),
    )(feat, aggs, wt, b2d)


def kernel(feat, edge_index, edge_affine, W, b):
    n, d = feat.shape
    dh = d // 2
    e = edge_index.shape[1]
    n_pad = ((n + 639) // 640) * 640
    unit = 16 * _CHUNK * 8 * _NPASS
    e_pad = ((e + unit - 1) // unit) * unit
    pad = e_pad - e

    src = jnp.concatenate([edge_index[0], jnp.zeros((pad,), jnp.int32)])
    dst = jnp.concatenate([edge_index[1], jnp.zeros((pad,), jnp.int32)])
    aff = jnp.concatenate([edge_affine, jnp.zeros((pad,), jnp.float32)])
    src2d = src.reshape(e_pad // _CHUNK, _CHUNK)
    dst2d = dst.reshape(e_pad // _CHUNK, _CHUNK)
    aff2d = aff.reshape(e_pad // _CHUNK, _CHUNK)
    feat_cat = jnp.concatenate([feat[:, :dh], feat[:, dh:]], axis=0)

    aggs = _sc_partial_agg(feat_cat, src2d, dst2d, aff2d, n, n_pad)
    return _tc_combine(feat, aggs, W.T, b[None, :])


# speedup vs baseline: 6.2445x; 1.1816x over previous
import functools

import jax
import jax.numpy as jnp
from jax import lax
from jax.experimental import pallas as pl
from jax.experimental.pallas import tpu as pltpu
from jax.experimental.pallas import tpu_sc as plsc

_CHUNK = 128
_NBUF = 4
_NPASS = 4


def _sc_partial_agg(feat2, src2d, dst2d, aff2d, n_pad):
    dh = feat2.shape[1]
    n_chunks = src2d.shape[0]
    cpt = n_chunks // 16
    cpp = cpt // _NPASS
    rpt = n_pad // 16
    mesh = plsc.VectorSubcoreMesh(core_axis_name="c", subcore_axis_name="s")

    @functools.partial(
        pl.kernel,
        out_type=jax.ShapeDtypeStruct((2, n_pad, dh), jnp.float32),
        mesh=mesh,
        compiler_params=pltpu.CompilerParams(use_tc_tiling_on_sc=False),
        scratch_types=[
            pltpu.VMEM((cpp, _CHUNK), jnp.int32),
            pltpu.VMEM((cpp, _CHUNK), jnp.int32),
            pltpu.VMEM((cpp, _CHUNK), jnp.float32),
            pltpu.VMEM((_NBUF, _CHUNK, dh), jnp.float32),
            pltpu.VMEM_SHARED((n_pad, dh), jnp.float32),
            pltpu.VMEM_SHARED((n_pad, dh), jnp.float32),
            pltpu.SemaphoreType.DMA((_NBUF,)),
            pltpu.SemaphoreType.DMA((_NBUF,)),
        ],
    )
    def k(feat_hbm, src_hbm, dst_hbm, aff_hbm, out_hbm,
          src_v, dst_v, aff_v, rows_v, fsp_sh, agg_sh, sg, ss):
        cid = lax.axis_index("c")
        sid = lax.axis_index("s")
        ngrp = dh // 16

        pltpu.sync_copy(feat_hbm.at[pl.ds(cid * n_pad + sid * rpt, rpt)],
                        fsp_sh.at[pl.ds(sid * rpt, rpt)])

        zbuf = rows_v.at[0]

        def zrow(r, carry):
            for g in range(ngrp):
                zbuf[r, pl.ds(g * 16, 16)] = jnp.zeros((16,), jnp.float32)
            return carry
        lax.fori_loop(0, _CHUNK, zrow, 0)
        for i in range(rpt // _CHUNK):
            pltpu.sync_copy(zbuf,
                            agg_sh.at[pl.ds(sid * rpt + i * _CHUNK, _CHUNK)])
        plsc.subcore_barrier()

        def _gather(j, b):
            pltpu.async_copy(fsp_sh.at[src_v.at[j]], rows_v.at[b], sg.at[b])

        def _scale(j, b):
            buf = rows_v.at[b]

            def edge_body(eg, c2):
                a16 = aff_v[j, pl.ds(eg * 16, 16)]
                for l in range(16):
                    av = jnp.full((16,), a16[l], jnp.float32)
                    e = eg * 16 + l
                    for g in range(ngrp):
                        sl = pl.ds(g * 16, 16)
                        buf[e, sl] = buf[e, sl] * av
                return c2
            lax.fori_loop(0, _CHUNK // 16, edge_body, 0)

        for p in range(_NPASS):
            base = sid * cpt + p * cpp
            pltpu.sync_copy(src_hbm.at[pl.ds(base, cpp)], src_v)
            pltpu.sync_copy(dst_hbm.at[pl.ds(base, cpp)], dst_v)
            pltpu.sync_copy(aff_hbm.at[pl.ds(base, cpp)], aff_v)

            _gather(0, 0)
            _gather(1, 1)

            def chunk_body(i, carry):
                for u in range(_NBUF):
                    j = i * _NBUF + u
                    b = u
                    bn = (u + 2) % _NBUF
                    pltpu.make_async_copy(fsp_sh.at[src_v.at[j]],
                                          rows_v.at[b], sg.at[b]).wait()
                    _scale(j, b)
                    pltpu.async_copy(rows_v.at[b], agg_sh.at[dst_v.at[j]],
                                     ss.at[b], add=True)

                    @pl.when(j >= 2)
                    def _drain():
                        pltpu.make_async_copy(rows_v.at[bn],
                                              agg_sh.at[dst_v.at[j]],
                                              ss.at[bn]).wait()

                    @pl.when(j + 2 < cpp)
                    def _refill():
                        _gather(j + 2, bn)
                return carry
            lax.fori_loop(0, cpp // _NBUF, chunk_body, 0)

            for b in (2, 3):
                pltpu.make_async_copy(rows_v.at[b], agg_sh.at[dst_v.at[0]],
                                      ss.at[b]).wait()

        plsc.subcore_barrier()
        pltpu.sync_copy(agg_sh.at[pl.ds(sid * rpt, rpt)],
                        out_hbm.at[cid, pl.ds(sid * rpt, rpt)])

    return k(feat2, src2d, dst2d, aff2d)


def _tc_combine(feat, aggs, wt, b2d):
    n, d = feat.shape
    dh = d // 2
    d_out = wt.shape[1]
    blk = 2000

    def body(x_ref, a_ref, w_ref, b_ref, o_ref):
        w1 = w_ref[pl.ds(0, d), :]
        w2a = w_ref[pl.ds(d, dh), :]
        w2b = w_ref[pl.ds(d + dh, dh), :]
        o_ref[...] = (
            jnp.dot(x_ref[...], w1, preferred_element_type=jnp.float32)
            + jnp.dot(a_ref[0], w2a, preferred_element_type=jnp.float32)
            + jnp.dot(a_ref[1], w2b, preferred_element_type=jnp.float32)
            + b_ref[...])

    return pl.pallas_call(
        body,
        grid=(n // blk,),
        in_specs=[
            pl.BlockSpec((blk, d), lambda i: (i, 0)),
            pl.BlockSpec((2, blk, dh), lambda i: (0, i, 0)),
            pl.BlockSpec((2 * d, d_out), lambda i: (0, 0)),
            pl.BlockSpec((1, d_out), lambda i: (0, 0)),
        ],
        out_specs=pl.BlockSpec((blk, d_out), lambda i: (i, 0)),
        out_shape=jax.ShapeDtypeStruct((n, d_out), jnp.float32),
    )(feat, aggs, wt, b2d)


def kernel(feat, edge_index, edge_affine, W, b):
    n, d = feat.shape
    dh = d // 2
    e = edge_index.shape[1]
    n_pad = ((n + 639) // 640) * 640
    unit = 16 * _CHUNK * 8 * _NPASS
    e_pad = ((e + unit - 1) // unit) * unit
    pad = e_pad - e

    src = jnp.concatenate([edge_index[0], jnp.zeros((pad,), jnp.int32)])
    dst = jnp.concatenate([edge_index[1], jnp.zeros((pad,), jnp.int32)])
    aff = jnp.concatenate([edge_affine, jnp.zeros((pad,), jnp.float32)])
    src2d = src.reshape(e_pad // _CHUNK, _CHUNK)
    dst2d = dst.reshape(e_pad // _CHUNK, _CHUNK)
    aff2d = aff.reshape(e_pad // _CHUNK, _CHUNK)
    rpad = jnp.zeros((n_pad - n, dh), jnp.float32)
    feat2 = jnp.concatenate([feat[:, :dh], rpad, feat[:, dh:], rpad], axis=0)

    aggs = _sc_partial_agg(feat2, src2d, dst2d, aff2d, n_pad)
    return _tc_combine(feat, aggs, W.T, b[None, :])
